# baseline (speedup 1.0000x reference)
import functools

import jax
import jax.numpy as jnp
from jax import lax
from jax.experimental import pallas as pl
from jax.experimental.pallas import tpu as pltpu
from jax.experimental.pallas import tpu_sc as plsc

_VOCAB = 1000
_VPAD = 1024
_BLOCK = 50
_N_EMBED = 64
_B = 1024
_T = 50
_ROWS = _B * _T

_INFO = plsc.get_sparse_core_info()
_NC = _INFO.num_cores
_NS = _INFO.num_subcores
_NW = _NC * _NS
_ROWS_PER_W = _ROWS // _NW
_CHUNK = 32
_NCHUNK = _ROWS_PER_W // _CHUNK
_NFULL = _VOCAB // 16
_TAILOFF = _VOCAB - 16


def _precompute_body(tok_emb_ref, pos_emb_ref, w_ref, b_ref,
                     fused_ref, posw_ref):
    w = w_ref[...]
    fused_ref[...] = (
        jnp.dot(tok_emb_ref[...], w, preferred_element_type=jnp.float32)
        + b_ref[...]
    )
    posw_ref[...] = jnp.dot(pos_emb_ref[...], w,
                            preferred_element_type=jnp.float32)


def _precompute(tok_emb, pos_emb, w_pad, b_pad):
    return pl.pallas_call(
        _precompute_body,
        out_shape=[
            jax.ShapeDtypeStruct((_VOCAB, _VPAD), jnp.float32),
            jax.ShapeDtypeStruct((_BLOCK, _VPAD), jnp.float32),
        ],
    )(tok_emb, pos_emb, w_pad, b_pad)


@functools.partial(
    pl.kernel,
    mesh=plsc.VectorSubcoreMesh(core_axis_name="c", subcore_axis_name="s"),
    out_type=jax.ShapeDtypeStruct((_ROWS, _VOCAB), jnp.float32),
    scratch_types=[
        pltpu.VMEM((_CHUNK,), jnp.int32),
        pltpu.VMEM((_CHUNK, _VPAD), jnp.float32),
        pltpu.VMEM((_CHUNK, _VOCAB), jnp.float32),
        pltpu.VMEM((_BLOCK, _VPAD), jnp.float32),
        pltpu.SemaphoreType.DMA,
    ],
)
def _sc_gather(tokens_hbm, fused_hbm, posw_hbm, out_hbm,
               idx_v, rows_v, rows_out, posw_v, sem):
    wid = lax.axis_index("s") * _NC + lax.axis_index("c")
    base = wid * _ROWS_PER_W
    pltpu.sync_copy(posw_hbm, posw_v)

    def chunk_body(c, carry):
        rowbase = base + c * _CHUNK
        pltpu.sync_copy(tokens_hbm.at[pl.ds(rowbase, _CHUNK)], idx_v)
        pltpu.async_copy(fused_hbm.at[idx_v], rows_v, sem).wait()

        def row_body(r, rcarry):
            t = lax.rem(rowbase + r, _BLOCK)
            for j in range(_NFULL):
                rows_out[r, pl.ds(16 * j, 16)] = (
                    rows_v[r, pl.ds(16 * j, 16)]
                    + posw_v[t, pl.ds(16 * j, 16)]
                )
            rows_out[r, pl.ds(_TAILOFF, 16)] = (
                rows_v[r, pl.ds(_TAILOFF, 16)]
                + posw_v[t, pl.ds(_TAILOFF, 16)]
            )
            return rcarry

        lax.fori_loop(0, _CHUNK, row_body, 0)
        pltpu.sync_copy(rows_out, out_hbm.at[pl.ds(rowbase, _CHUNK)])
        return carry

    lax.fori_loop(0, _NCHUNK, chunk_body, 0)


def kernel(tokens, tok_emb, pos_emb, W, b):
    w_pad = jnp.pad(W, ((0, 0), (0, _VPAD - _VOCAB)))
    b_pad = jnp.pad(b, (0, _VPAD - _VOCAB)).reshape(1, _VPAD)
    fused, posw = _precompute(tok_emb, pos_emb, w_pad, b_pad)
    tokens_flat = tokens.reshape(_ROWS).astype(jnp.int32)
    out = _sc_gather(tokens_flat, fused, posw)
    return out.reshape(_B, _T, _VOCAB)


# speedup vs baseline: 1.2161x; 1.2161x over previous
import functools

import jax
import jax.numpy as jnp
from jax import lax
from jax.experimental import pallas as pl
from jax.experimental.pallas import tpu as pltpu
from jax.experimental.pallas import tpu_sc as plsc

_VOCAB = 1000
_BLOCK = 50
_N_EMBED = 64
_B = 1024
_T = 50

_INFO = plsc.get_sparse_core_info()
_NC = _INFO.num_cores
_NS = _INFO.num_subcores
_NW = _NC * _NS
_B_PER_W = _B // _NW
_NFULL = _VOCAB // 16
_TAILOFF = _VOCAB - 16


def _precompute_body(tok_emb_ref, pos_emb_ref, w_ref, b_ref,
                     fused_ref, posw_ref, pwtail_ref):
    w = w_ref[...]
    fused_ref[...] = (
        jnp.dot(tok_emb_ref[...], w, preferred_element_type=jnp.float32)
        + b_ref[...]
    )
    pw = jnp.dot(pos_emb_ref[...], w, preferred_element_type=jnp.float32)
    posw_ref[...] = pw
    pwtail_ref[...] = jnp.concatenate(
        [jnp.zeros((_BLOCK, 8), jnp.float32), pw[:, _NFULL * 16:_VOCAB]],
        axis=1,
    )


def _precompute(tok_emb, pos_emb, w, b):
    return pl.pallas_call(
        _precompute_body,
        out_shape=[
            jax.ShapeDtypeStruct((_VOCAB, _VOCAB), jnp.float32),
            jax.ShapeDtypeStruct((_BLOCK, _VOCAB), jnp.float32),
            jax.ShapeDtypeStruct((_BLOCK, 16), jnp.float32),
        ],
    )(tok_emb, pos_emb, w, b.reshape(1, _VOCAB))


@functools.partial(
    pl.kernel,
    mesh=plsc.VectorSubcoreMesh(core_axis_name="c", subcore_axis_name="s"),
    out_type=jax.ShapeDtypeStruct((_B, _T, _VOCAB), jnp.float32),
    scratch_types=[
        pltpu.VMEM((_B_PER_W, _T), jnp.int32),
        pltpu.VMEM((_T, _VOCAB), jnp.float32),
        pltpu.VMEM((_BLOCK, _VOCAB), jnp.float32),
        pltpu.VMEM((_BLOCK, 16), jnp.float32),
        pltpu.SemaphoreType.DMA,
    ],
    compiler_params=pltpu.CompilerParams(use_tc_tiling_on_sc=False),
)
def _sc_gather(tokens_hbm, fused_hbm, posw_hbm, pwtail_hbm, out_hbm,
               idx_all, rows_v, posw_v, pwtail_v, sem):
    wid = lax.axis_index("s") * _NC + lax.axis_index("c")
    bbase = wid * _B_PER_W
    pltpu.sync_copy(posw_hbm, posw_v)
    pltpu.sync_copy(pwtail_hbm, pwtail_v)
    pltpu.sync_copy(tokens_hbm.at[pl.ds(bbase, _B_PER_W)], idx_all)

    def chunk_body(c, carry):
        bi = bbase + c
        pltpu.async_copy(fused_hbm.at[idx_all.at[c]], rows_v, sem).wait()

        def row_body(r, rcarry):
            for j in range(_NFULL):
                plsc.addupdate(rows_v.at[r, pl.ds(16 * j, 16)],
                               posw_v[r, pl.ds(16 * j, 16)])
            rows_v[r, pl.ds(_TAILOFF, 16)] += pwtail_v[r, :]
            return rcarry

        lax.fori_loop(0, _T, row_body, 0)
        pltpu.sync_copy(rows_v, out_hbm.at[bi])
        return carry

    lax.fori_loop(0, _B_PER_W, chunk_body, 0)


def kernel(tokens, tok_emb, pos_emb, W, b):
    fused, posw, pwtail = _precompute(tok_emb, pos_emb, W, b)
    return _sc_gather(tokens.astype(jnp.int32), fused, posw, pwtail)


# speedup vs baseline: 1.3830x; 1.1372x over previous
import functools

import jax
import jax.numpy as jnp
from jax import lax
from jax.experimental import pallas as pl
from jax.experimental.pallas import tpu as pltpu
from jax.experimental.pallas import tpu_sc as plsc

_VOCAB = 1000
_VPAD = 1024
_BLOCK = 50
_TPAD = 56
_PPAD = 64
_N_EMBED = 64
_B = 1024
_T = 50

_INFO = plsc.get_sparse_core_info()
_NC = _INFO.num_cores
_NS = _INFO.num_subcores
_NW = _NC * _NS
_B_PER_W = _B // _NW
_NPAIR = _B_PER_W // 2

_BITES = ((0, 16, 16), (16, 16, 16), (32, 16, 16), (48, 2, 8))


def _precompute_body(tok_emb_ref, pos_emb_ref, w_ref, b_ref,
                     fused_ref, posw_ref):
    w = w_ref[...]
    fused_ref[...] = (
        jnp.dot(tok_emb_ref[...], w, preferred_element_type=jnp.float32)
        + b_ref[...]
    )
    posw_ref[...] = jnp.dot(pos_emb_ref[...], w,
                            preferred_element_type=jnp.float32)


def _precompute(tok_emb, pos_emb_pad, w_pad, b_pad):
    return pl.pallas_call(
        _precompute_body,
        out_shape=[
            jax.ShapeDtypeStruct((_VOCAB, _VPAD), jnp.float32),
            jax.ShapeDtypeStruct((_PPAD, _VPAD), jnp.float32),
        ],
    )(tok_emb, pos_emb_pad, w_pad, b_pad)


@functools.partial(
    pl.kernel,
    mesh=plsc.VectorSubcoreMesh(core_axis_name="c", subcore_axis_name="s"),
    out_type=jax.ShapeDtypeStruct((_B, _T, _VOCAB), jnp.float32),
    scratch_types=[
        pltpu.VMEM((_B_PER_W, _TPAD), jnp.int32),
        pltpu.VMEM((16, _VPAD), jnp.float32),
        pltpu.VMEM((16, _VPAD), jnp.float32),
        pltpu.VMEM((16, _VPAD), jnp.float32),
        pltpu.VMEM((16, _VOCAB), jnp.float32),
        pltpu.VMEM((16, _VOCAB), jnp.float32),
        pltpu.SemaphoreType.DMA,
        pltpu.SemaphoreType.DMA,
        pltpu.SemaphoreType.DMA,
        pltpu.SemaphoreType.DMA,
    ],
)
def _sc_gather(tokens_hbm, fused_hbm, posw_hbm, out_hbm,
               idx_all, poswb, gb0, gb1, ob0, ob1, gs0, gs1, ws0, ws1):
    wid = lax.axis_index("s") * _NC + lax.axis_index("c")
    bbase = wid * _B_PER_W
    pltpu.sync_copy(tokens_hbm.at[pl.ds(bbase, _B_PER_W)], idx_all)

    for off, n, ng in _BITES:
        pltpu.sync_copy(posw_hbm.at[pl.ds(off, 16)], poswb)
        gbufs = (gb0, gb1)
        obufs = (ob0, ob1)
        gsems = (gs0, gs1)
        wsems = (ws0, ws1)

        def g_dst(gb):
            return gb if ng == 16 else gb.at[pl.ds(0, ng)]

        def o_src(ob):
            return ob if n == 16 else ob.at[pl.ds(0, n)]

        def g_issue(c, gb, gs):
            pltpu.async_copy(
                fused_hbm.at[idx_all.at[c, pl.ds(off, ng)]], g_dst(gb), gs)

        g_issue(0, gb0, gs0)
        g_issue(1, gb1, gs1)

        def pair_body(p, carry):
            for par in (0, 1):
                cc = 2 * p + par
                gb, ob, gs, ws = gbufs[par], obufs[par], gsems[par], wsems[par]
                bi = bbase + cc
                pltpu.make_async_copy(
                    fused_hbm.at[pl.ds(0, ng)], g_dst(gb), gs).wait()
                @pl.when(p > 0)
                def _():
                    pltpu.make_async_copy(
                        o_src(ob), out_hbm.at[bi, pl.ds(off, n)], ws).wait()

                def row_body(r, rcarry):
                    for j in range(_VOCAB // 16):
                        ob[r, pl.ds(16 * j, 16)] = (
                            gb[r, pl.ds(16 * j, 16)]
                            + poswb[r, pl.ds(16 * j, 16)]
                        )
                    ob[r, pl.ds(_VOCAB - 16, 16)] = (
                        gb[r, pl.ds(_VOCAB - 16, 16)]
                        + poswb[r, pl.ds(_VOCAB - 16, 16)]
                    )
                    return rcarry

                lax.fori_loop(0, n, row_body, 0)
                pltpu.async_copy(o_src(ob), out_hbm.at[bi, pl.ds(off, n)], ws)

                @pl.when(p < _NPAIR - 1)
                def _():
                    g_issue(cc + 2, gb, gs)
            return carry

        lax.fori_loop(0, _NPAIR, pair_body, 0)
        for par in (0, 1):
            pltpu.make_async_copy(
                o_src(obufs[par]),
                out_hbm.at[bbase + _B_PER_W - 2 + par, pl.ds(off, n)],
                wsems[par]).wait()


def kernel(tokens, tok_emb, pos_emb, W, b):
    w_pad = jnp.pad(W, ((0, 0), (0, _VPAD - _VOCAB)))
    b_pad = jnp.pad(b, (0, _VPAD - _VOCAB)).reshape(1, _VPAD)
    pos_emb_pad = jnp.pad(pos_emb, ((0, _PPAD - _BLOCK), (0, 0)))
    tokens_pad = jnp.pad(tokens.astype(jnp.int32),
                         ((0, 0), (0, _TPAD - _T)))
    fused, posw = _precompute(tok_emb, pos_emb_pad, w_pad, b_pad)
    return _sc_gather(tokens_pad, fused, posw)


# speedup vs baseline: 1.6737x; 1.2103x over previous
import functools

import jax
import jax.numpy as jnp
from jax import lax
from jax.experimental import pallas as pl
from jax.experimental.pallas import tpu as pltpu
from jax.experimental.pallas import tpu_sc as plsc

_VOCAB = 1000
_VPAD = 1024
_BLOCK = 50
_TPAD = 64
_N_EMBED = 64
_B = 1024
_T = 50

_INFO = plsc.get_sparse_core_info()
_NC = _INFO.num_cores
_NS = _INFO.num_subcores
_NW = _NC * _NS
_NV = _VPAD // _NW
_NPAIR = _T // 2


def _precompute_body(tok_emb_ref, pos_emb_ref, w_ref, b_ref,
                     fusedt_ref, poswt_ref):
    w = w_ref[...]
    dn = (((0,), (1,)), ((), ()))
    fusedt_ref[...] = (
        lax.dot_general(w, tok_emb_ref[...], dn,
                        preferred_element_type=jnp.float32)
        + b_ref[...]
    )
    poswt_ref[...] = lax.dot_general(w, pos_emb_ref[...], dn,
                                     preferred_element_type=jnp.float32)


def _precompute(tok_emb_pad, pos_emb_pad, w_pad, b_col):
    return pl.pallas_call(
        _precompute_body,
        out_shape=[
            jax.ShapeDtypeStruct((_VPAD, _VPAD), jnp.float32),
            jax.ShapeDtypeStruct((_VPAD, _TPAD), jnp.float32),
        ],
    )(tok_emb_pad, pos_emb_pad, w_pad, b_col)


@functools.partial(
    pl.kernel,
    mesh=plsc.VectorSubcoreMesh(core_axis_name="c", subcore_axis_name="s"),
    out_type=jax.ShapeDtypeStruct((_T, _VOCAB, _B), jnp.float32),
    scratch_types=[
        pltpu.VMEM((_NV, _VPAD), jnp.float32),
        pltpu.VMEM((_NV, _TPAD), jnp.float32),
        pltpu.VMEM((_B,), jnp.int32),
        pltpu.VMEM((_B,), jnp.int32),
        pltpu.VMEM((_NV, _B), jnp.float32),
        pltpu.VMEM((_NV, _B), jnp.float32),
        pltpu.SemaphoreType.DMA,
        pltpu.SemaphoreType.DMA,
        pltpu.SemaphoreType.DMA,
        pltpu.SemaphoreType.DMA,
    ],
    compiler_params=pltpu.CompilerParams(needs_layout_passes=False),
)
def _sc_logits(tokenst_hbm, fusedt_hbm, poswt_hbm, outt_hbm,
               fusedt_v, poswt_v, tk0, tk1, ob0, ob1, ts0, ts1, ws0, ws1):
    wid = lax.axis_index("s") * _NC + lax.axis_index("c")
    vbase = wid * _NV
    pltpu.sync_copy(fusedt_hbm.at[pl.ds(vbase, _NV)], fusedt_v)
    pltpu.sync_copy(poswt_hbm.at[pl.ds(vbase, _NV)], poswt_v)
    tks = (tk0, tk1)
    obs = (ob0, ob1)
    tss = (ts0, ts1)
    wss = (ws0, ws1)
    pltpu.async_copy(tokenst_hbm.at[0], tk0, ts0)
    pltpu.async_copy(tokenst_hbm.at[1], tk1, ts1)

    def write_out(ob, t, ws):
        @pl.when(wid < _NW - 1)
        def _():
            pltpu.async_copy(ob, outt_hbm.at[t, pl.ds(vbase, _NV)], ws)

        @pl.when(wid == _NW - 1)
        def _():
            pltpu.async_copy(ob.at[pl.ds(0, 8)],
                             outt_hbm.at[t, pl.ds(vbase, 8)], ws)

    def drain_write(ob, t, ws):
        @pl.when(wid < _NW - 1)
        def _():
            pltpu.make_async_copy(
                ob, outt_hbm.at[t, pl.ds(vbase, _NV)], ws).wait()

        @pl.when(wid == _NW - 1)
        def _():
            pltpu.make_async_copy(
                ob.at[pl.ds(0, 8)], outt_hbm.at[t, pl.ds(vbase, 8)], ws).wait()

    def pair_body(p, carry):
        for par in (0, 1):
            t = 2 * p + par
            tk, ob, ts, ws = tks[par], obs[par], tss[par], wss[par]
            pltpu.make_async_copy(tokenst_hbm.at[t], tk, ts).wait()

            @pl.when(p > 0)
            def _():
                drain_write(ob, t, ws)

            tvec = jnp.zeros((16,), jnp.int32) + t
            psplat = [
                plsc.load_gather(
                    poswt_v,
                    [jnp.full((16,), vl, jnp.int32), tvec])
                for vl in range(_NV)
            ]

            def bc_body(bc, bcarry):
                tok = tk[pl.ds(16 * bc, 16)]
                for vl in range(_NV):
                    vals = plsc.load_gather(
                        fusedt_v, [jnp.full((16,), vl, jnp.int32), tok])
                    ob[vl, pl.ds(16 * bc, 16)] = vals + psplat[vl]
                return bcarry

            lax.fori_loop(0, _B // 16, bc_body, 0)
            write_out(ob, t, ws)

            @pl.when(p < _NPAIR - 1)
            def _():
                pltpu.async_copy(tokenst_hbm.at[t + 2], tk, ts)
        return carry

    lax.fori_loop(0, _NPAIR, pair_body, 0)
    for par in (0, 1):
        drain_write(obs[par], _T - 2 + par, wss[par])


def kernel(tokens, tok_emb, pos_emb, W, b):
    w_pad = jnp.pad(W, ((0, 0), (0, _VPAD - _VOCAB)))
    b_col = jnp.pad(b, (0, _VPAD - _VOCAB)).reshape(_VPAD, 1)
    tok_emb_pad = jnp.pad(tok_emb, ((0, _VPAD - _VOCAB), (0, 0)))
    pos_emb_pad = jnp.pad(pos_emb, ((0, _TPAD - _BLOCK), (0, 0)))
    tokenst = tokens.astype(jnp.int32).T
    fusedt, poswt = _precompute(tok_emb_pad, pos_emb_pad, w_pad, b_col)
    outt = _sc_logits(tokenst, fusedt, poswt)
    return jnp.transpose(outt, (2, 0, 1))
